# baseline (speedup 1.0000x reference)
import functools

import jax
import jax.numpy as jnp
from jax import lax
from jax.experimental import pallas as pl
from jax.experimental.pallas import tpu as pltpu
from jax.experimental.pallas import tpu_sc as plsc

N = 10000
E = 320000
D = 128
EPS = 1e-5

NCORES = 2
NSUB = 16
NW = NCORES * NSUB
NP = 10112
ROWS_PER_SUB = NP // NSUB
CHUNK = 128
CPW = 80
E_PAD = NW * CHUNK * CPW
DEG_W = 16

ROW_BLK = 2000


def _proj_rows(x):
    d2 = jnp.sum(x * x, axis=1, keepdims=True)
    norm = jnp.sqrt(d2 + 1e-15)
    max_norm = 1.0 - EPS
    scale = jnp.where(norm > max_norm, max_norm / norm, jnp.ones_like(norm))
    return x * scale


def _pre_body(x_ref, w_ref, b_ref, h_ref, *, project):
    x = x_ref[...]
    if project:
        x = _proj_rows(x)
    h_ref[...] = (
        jnp.dot(x, w_ref[...], preferred_element_type=jnp.float32,
                precision=lax.Precision.HIGHEST)
        + b_ref[...]
    )


def _tc_pre(x, w, b, project):
    return pl.pallas_call(
        functools.partial(_pre_body, project=project),
        grid=(N // ROW_BLK,),
        in_specs=[
            pl.BlockSpec((ROW_BLK, D), lambda i: (i, 0)),
            pl.BlockSpec((D, D), lambda i: (0, 0)),
            pl.BlockSpec((1, D), lambda i: (0, 0)),
        ],
        out_specs=pl.BlockSpec((ROW_BLK, D), lambda i: (i, 0)),
        out_shape=jax.ShapeDtypeStruct((N, D), jnp.float32),
    )(x, w, b.reshape(1, D))


def _post_body(h_ref, p_ref, d_ref, o_ref, *, activation):
    h = h_ref[...]
    agg = p_ref[0] + p_ref[1]
    deg = d_ref[0][:, 0:1] + d_ref[1][:, 0:1]
    deg = jnp.maximum(deg, 1.0)
    out = _proj_rows(0.5 * (h + agg / deg))
    if activation:
        d2 = jnp.sum(out * out, axis=1, keepdims=True)
        denom = jnp.clip(1.0 - d2, 1e-7, None)
        arg = jnp.clip(1.0 + 2.0 * d2 / denom, 1.0 + 1e-7, None)
        nrm = jnp.log(arg + jnp.sqrt(arg * arg - 1.0))
        sig = 1.0 / (1.0 + jnp.exp(-nrm))
        out = _proj_rows(out * sig)
    o_ref[...] = out


def _tc_post(h, agg_p, deg_p, activation):
    return pl.pallas_call(
        functools.partial(_post_body, activation=activation),
        grid=(N // ROW_BLK,),
        in_specs=[
            pl.BlockSpec((ROW_BLK, D), lambda i: (i, 0)),
            pl.BlockSpec((NCORES, ROW_BLK, D), lambda i: (0, i, 0)),
            pl.BlockSpec((NCORES, ROW_BLK, DEG_W), lambda i: (0, i, 0)),
        ],
        out_specs=pl.BlockSpec((ROW_BLK, D), lambda i: (i, 0)),
        out_shape=jax.ShapeDtypeStruct((N, D), jnp.float32),
    )(h, agg_p, deg_p)


def _sc_agg_body(h_hbm, ed_hbm, zagg_hbm, zdeg_hbm, ones_hbm,
                 agg_out, deg_out, idx_v, rows_v, ones_v,
                 acc_sh, dacc_sh, isem, gsem, asem, dsem):
    c = lax.axis_index("c")
    s = lax.axis_index("s")
    wid = c * NSUB + s
    r0 = s * ROWS_PER_SUB
    pltpu.sync_copy(zagg_hbm, acc_sh.at[pl.ds(r0, ROWS_PER_SUB)])
    pltpu.sync_copy(zdeg_hbm, dacc_sh.at[pl.ds(r0, ROWS_PER_SUB)])
    pltpu.sync_copy(ones_hbm, ones_v)
    plsc.subcore_barrier()

    t0 = wid * CPW
    pltpu.sync_copy(ed_hbm.at[t0], idx_v.at[0])
    pltpu.async_copy(h_hbm.at[idx_v.at[0].at[0]], rows_v.at[0], gsem)
    pltpu.async_copy(ed_hbm.at[t0 + 1], idx_v.at[1], isem)

    def half(e, b, nb):
        pltpu.make_async_copy(h_hbm.at[idx_v.at[b].at[0]],
                              rows_v.at[b], gsem).wait()
        pltpu.make_async_copy(ed_hbm.at[e + 1], idx_v.at[nb], isem).wait()
        pltpu.async_copy(h_hbm.at[idx_v.at[nb].at[0]], rows_v.at[nb], gsem)
        pltpu.async_copy(rows_v.at[b], acc_sh.at[idx_v.at[b].at[1]],
                         asem, add=True)
        pltpu.async_copy(ones_v, dacc_sh.at[idx_v.at[b].at[1]],
                         dsem, add=True)
        pltpu.make_async_copy(rows_v.at[b], acc_sh.at[idx_v.at[b].at[1]],
                              asem).wait()
        pltpu.make_async_copy(ones_v, dacc_sh.at[idx_v.at[b].at[1]],
                              dsem).wait()
        pltpu.async_copy(ed_hbm.at[e + 2], idx_v.at[b], isem)

    def body(jj, carry):
        e = t0 + 2 * jj
        half(e, 0, 1)
        half(e + 1, 1, 0)
        return carry

    lax.fori_loop(0, CPW // 2, body, 0)
    pltpu.make_async_copy(h_hbm.at[idx_v.at[0].at[0]], rows_v.at[0],
                          gsem).wait()
    pltpu.make_async_copy(ed_hbm.at[t0], idx_v.at[1], isem).wait()
    plsc.subcore_barrier()
    out_base = c * NP + r0
    pltpu.sync_copy(acc_sh.at[pl.ds(r0, ROWS_PER_SUB)],
                    agg_out.at[pl.ds(out_base, ROWS_PER_SUB)])
    pltpu.sync_copy(dacc_sh.at[pl.ds(r0, ROWS_PER_SUB)],
                    deg_out.at[pl.ds(out_base, ROWS_PER_SUB)])


@functools.cache
def _sc_agg():
    mesh = plsc.VectorSubcoreMesh(core_axis_name="c", subcore_axis_name="s",
                                  num_cores=NCORES, num_subcores=NSUB)
    return pl.kernel(
        _sc_agg_body,
        out_type=(
            jax.ShapeDtypeStruct((NCORES * NP, D), jnp.float32),
            jax.ShapeDtypeStruct((NCORES * NP, DEG_W), jnp.float32),
        ),
        mesh=mesh,
        compiler_params=pltpu.CompilerParams(use_tc_tiling_on_sc=False),
        scratch_types=[
            pltpu.VMEM((2, 2, CHUNK), jnp.int32),
            pltpu.VMEM((2, CHUNK, D), jnp.float32),
            pltpu.VMEM((CHUNK, DEG_W), jnp.float32),
            pltpu.VMEM_SHARED((NP, D), jnp.float32),
            pltpu.VMEM_SHARED((NP, DEG_W), jnp.float32),
            pltpu.SemaphoreType.DMA,
            pltpu.SemaphoreType.DMA,
            pltpu.SemaphoreType.DMA,
            pltpu.SemaphoreType.DMA,
        ],
    )


def kernel(x, edge_index, W1, b1, W2, b2):
    src = edge_index[0].astype(jnp.int32)
    dst = edge_index[1].astype(jnp.int32)
    pad = E_PAD - E
    src = jnp.concatenate([src, jnp.zeros((pad,), jnp.int32)])
    dst = jnp.concatenate([dst, jnp.full((pad,), N, jnp.int32)])
    ed = jnp.stack([src.reshape(-1, CHUNK), dst.reshape(-1, CHUNK)], axis=1)
    ed = jnp.concatenate([ed, ed[:2]], axis=0)
    zagg = jnp.zeros((ROWS_PER_SUB, D), jnp.float32)
    zdeg = jnp.zeros((ROWS_PER_SUB, DEG_W), jnp.float32)
    ones = jnp.ones((CHUNK, DEG_W), jnp.float32)

    h1 = _tc_pre(x, W1, b1, project=True)
    agg1, deg1 = _sc_agg()(h1, ed, zagg, zdeg, ones)
    y1 = _tc_post(h1, agg1.reshape(NCORES, NP, D),
                  deg1.reshape(NCORES, NP, DEG_W), activation=True)
    h2 = _tc_pre(y1, W2, b2, project=False)
    agg2, deg2 = _sc_agg()(h2, ed, zagg, zdeg, ones)
    out = _tc_post(h2, agg2.reshape(NCORES, NP, D),
                   deg2.reshape(NCORES, NP, DEG_W), activation=False)
    return out


# speedup vs baseline: 1.0189x; 1.0189x over previous
import functools

import jax
import jax.numpy as jnp
from jax import lax
from jax.experimental import pallas as pl
from jax.experimental.pallas import tpu as pltpu
from jax.experimental.pallas import tpu_sc as plsc

N = 10000
E = 320000
D = 128
EPS = 1e-5

NCORES = 2
NSUB = 16
NW = NCORES * NSUB
NP = 10016
ROWS_PER_SUB = NP // NSUB
CHUNK = 128
CPW = 80
E_PAD = NW * CHUNK * CPW
DEG_W = 16

ROW_BLK = 2000


def _proj_rows(x):
    d2 = jnp.sum(x * x, axis=1, keepdims=True)
    norm = jnp.sqrt(d2 + 1e-15)
    max_norm = 1.0 - EPS
    scale = jnp.where(norm > max_norm, max_norm / norm, jnp.ones_like(norm))
    return x * scale


def _pre_body(x_ref, w_ref, b_ref, h_ref, *, project):
    x = x_ref[...]
    if project:
        x = _proj_rows(x)
    h_ref[...] = (
        jnp.dot(x, w_ref[...], preferred_element_type=jnp.float32,
                precision=lax.Precision.HIGHEST)
        + b_ref[...]
    )


def _tc_pre(x, w, b, project):
    return pl.pallas_call(
        functools.partial(_pre_body, project=project),
        grid=(N // ROW_BLK,),
        in_specs=[
            pl.BlockSpec((ROW_BLK, D), lambda i: (i, 0)),
            pl.BlockSpec((D, D), lambda i: (0, 0)),
            pl.BlockSpec((1, D), lambda i: (0, 0)),
        ],
        out_specs=pl.BlockSpec((ROW_BLK, D), lambda i: (i, 0)),
        out_shape=jax.ShapeDtypeStruct((N, D), jnp.float32),
    )(x, w, b.reshape(1, D))


def _post_body(h_ref, p_ref, d_ref, o_ref, *, activation):
    h = h_ref[...]
    agg = p_ref[0] + p_ref[1]
    deg = d_ref[0][:, 0:1] + d_ref[1][:, 0:1]
    deg = jnp.maximum(deg, 1.0)
    out = _proj_rows(0.5 * (h + agg / deg))
    if activation:
        d2 = jnp.sum(out * out, axis=1, keepdims=True)
        denom = jnp.clip(1.0 - d2, 1e-7, None)
        arg = jnp.clip(1.0 + 2.0 * d2 / denom, 1.0 + 1e-7, None)
        nrm = jnp.log(arg + jnp.sqrt(arg * arg - 1.0))
        sig = 1.0 / (1.0 + jnp.exp(-nrm))
        out = _proj_rows(out * sig)
    o_ref[...] = out


def _tc_post(h, agg_p, deg_p, activation):
    return pl.pallas_call(
        functools.partial(_post_body, activation=activation),
        grid=(N // ROW_BLK,),
        in_specs=[
            pl.BlockSpec((ROW_BLK, D), lambda i: (i, 0)),
            pl.BlockSpec((NCORES, ROW_BLK, D), lambda i: (0, i, 0)),
            pl.BlockSpec((NCORES, ROW_BLK, DEG_W), lambda i: (0, i, 0)),
        ],
        out_specs=pl.BlockSpec((ROW_BLK, D), lambda i: (i, 0)),
        out_shape=jax.ShapeDtypeStruct((N, D), jnp.float32),
    )(h, agg_p, deg_p)


def _sc_agg_body(h_hbm, ed_hbm, zagg_hbm, zdeg_hbm, ones_hbm,
                 agg_out, deg_out, idx_v, rows_v, ones_v,
                 acc_sh, dacc_sh, isem, gsem, asem, dsem):
    c = lax.axis_index("c")
    s = lax.axis_index("s")
    wid = c * NSUB + s
    r0 = s * ROWS_PER_SUB
    pltpu.sync_copy(zagg_hbm, acc_sh.at[pl.ds(r0, ROWS_PER_SUB)])
    pltpu.sync_copy(zdeg_hbm, dacc_sh.at[pl.ds(r0, ROWS_PER_SUB)])
    pltpu.sync_copy(ones_hbm, ones_v)
    plsc.subcore_barrier()

    t0 = wid * CPW

    def half(e, br, bi, skip_scwait):
        br1, bi1, bip, bld = 1 - br, (bi + 1) % 4, (bi - 1) % 4, (bi + 3) % 4
        pltpu.make_async_copy(h_hbm.at[idx_v.at[bi].at[0]],
                              rows_v.at[br], gsem).wait()
        pltpu.async_copy(rows_v.at[br], acc_sh.at[idx_v.at[bi].at[1]],
                         asem, add=True)
        pltpu.async_copy(ones_v, dacc_sh.at[idx_v.at[bi].at[1]],
                         dsem, add=True)
        if not skip_scwait:
            pltpu.make_async_copy(rows_v.at[br1],
                                  acc_sh.at[idx_v.at[bip].at[1]],
                                  asem).wait()
            pltpu.make_async_copy(ones_v, dacc_sh.at[idx_v.at[bip].at[1]],
                                  dsem).wait()
        pltpu.make_async_copy(ed_hbm.at[e + 1], idx_v.at[bi1], isem).wait()
        pltpu.async_copy(h_hbm.at[idx_v.at[bi1].at[0]], rows_v.at[br1], gsem)
        pltpu.async_copy(ed_hbm.at[e + 3], idx_v.at[bld], isem)

    pltpu.sync_copy(ed_hbm.at[t0], idx_v.at[0])
    pltpu.async_copy(ed_hbm.at[t0 + 1], idx_v.at[1], isem)
    pltpu.async_copy(ed_hbm.at[t0 + 2], idx_v.at[2], isem)
    pltpu.async_copy(h_hbm.at[idx_v.at[0].at[0]], rows_v.at[0], gsem)
    half(t0 + 0, 0, 0, True)
    half(t0 + 1, 1, 1, False)
    half(t0 + 2, 0, 2, False)
    half(t0 + 3, 1, 3, False)

    def body(jj, carry):
        e0 = t0 + 4 * jj
        half(e0 + 0, 0, 0, False)
        half(e0 + 1, 1, 1, False)
        half(e0 + 2, 0, 2, False)
        half(e0 + 3, 1, 3, False)
        return carry

    lax.fori_loop(1, CPW // 4, body, 0)
    pltpu.make_async_copy(rows_v.at[1], acc_sh.at[idx_v.at[3].at[1]],
                          asem).wait()
    pltpu.make_async_copy(ones_v, dacc_sh.at[idx_v.at[3].at[1]],
                          dsem).wait()
    pltpu.make_async_copy(h_hbm.at[idx_v.at[0].at[0]], rows_v.at[0],
                          gsem).wait()
    pltpu.make_async_copy(ed_hbm.at[t0], idx_v.at[1], isem).wait()
    pltpu.make_async_copy(ed_hbm.at[t0], idx_v.at[2], isem).wait()
    plsc.subcore_barrier()
    out_base = c * NP + r0
    pltpu.sync_copy(acc_sh.at[pl.ds(r0, ROWS_PER_SUB)],
                    agg_out.at[pl.ds(out_base, ROWS_PER_SUB)])
    pltpu.sync_copy(dacc_sh.at[pl.ds(r0, ROWS_PER_SUB)],
                    deg_out.at[pl.ds(out_base, ROWS_PER_SUB)])


@functools.cache
def _sc_agg():
    mesh = plsc.VectorSubcoreMesh(core_axis_name="c", subcore_axis_name="s",
                                  num_cores=NCORES, num_subcores=NSUB)
    return pl.kernel(
        _sc_agg_body,
        out_type=(
            jax.ShapeDtypeStruct((NCORES * NP, D), jnp.float32),
            jax.ShapeDtypeStruct((NCORES * NP, DEG_W), jnp.float32),
        ),
        mesh=mesh,
        compiler_params=pltpu.CompilerParams(use_tc_tiling_on_sc=False),
        scratch_types=[
            pltpu.VMEM((4, 2, CHUNK), jnp.int32),
            pltpu.VMEM((2, CHUNK, D), jnp.float32),
            pltpu.VMEM((CHUNK, DEG_W), jnp.float32),
            pltpu.VMEM_SHARED((NP, D), jnp.float32),
            pltpu.VMEM_SHARED((NP, DEG_W), jnp.float32),
            pltpu.SemaphoreType.DMA,
            pltpu.SemaphoreType.DMA,
            pltpu.SemaphoreType.DMA,
            pltpu.SemaphoreType.DMA,
        ],
    )


def kernel(x, edge_index, W1, b1, W2, b2):
    src = edge_index[0].astype(jnp.int32)
    dst = edge_index[1].astype(jnp.int32)
    pad = E_PAD - E
    src = jnp.concatenate([src, jnp.zeros((pad,), jnp.int32)])
    dst = jnp.concatenate([dst, jnp.full((pad,), N, jnp.int32)])
    ed = jnp.stack([src.reshape(-1, CHUNK), dst.reshape(-1, CHUNK)], axis=1)
    ed = jnp.concatenate([ed, ed[:3]], axis=0)
    zagg = jnp.zeros((ROWS_PER_SUB, D), jnp.float32)
    zdeg = jnp.zeros((ROWS_PER_SUB, DEG_W), jnp.float32)
    ones = jnp.ones((CHUNK, DEG_W), jnp.float32)

    h1 = _tc_pre(x, W1, b1, project=True)
    agg1, deg1 = _sc_agg()(h1, ed, zagg, zdeg, ones)
    y1 = _tc_post(h1, agg1.reshape(NCORES, NP, D),
                  deg1.reshape(NCORES, NP, DEG_W), activation=True)
    h2 = _tc_pre(y1, W2, b2, project=False)
    agg2, deg2 = _sc_agg()(h2, ed, zagg, zdeg, ones)
    out = _tc_post(h2, agg2.reshape(NCORES, NP, D),
                   deg2.reshape(NCORES, NP, DEG_W), activation=False)
    return out
